# baseline (speedup 1.0000x reference)
import jax
import jax.numpy as jnp
from jax.experimental import pallas as pl
from jax.experimental.pallas import tpu as pltpu

LAYERS = 6
HEADS = 8
LH = LAYERS * HEADS
L = 2048
T = 512


SCAN_BLOCK = 2
NSTREAM = 3
PART = LH // NSTREAM


def _scan_kernel(*refs):
    ins = refs[:NSTREAM]
    outs = refs[NSTREAM:]
    for x_ref, s_ref in zip(ins, outs):
        rmax = jnp.max(x_ref[...], axis=-1, keepdims=True)
        s_ref[...] = jnp.mean(rmax, axis=1, keepdims=True)


def _finalize_kernel(widx_ref, x_ref, score_ref, dur_ref, focus_ref):
    del widx_ref
    x = x_ref[0]
    rmax = jnp.max(x, axis=-1, keepdims=True)
    iota_t = jax.lax.broadcasted_iota(jnp.int32, (L, T), 1)
    ridx = jnp.min(jnp.where(x == rmax, iota_t, T), axis=-1, keepdims=True)
    eq = (ridx == iota_t).astype(jnp.int32)
    dur_ref[0, :] = jnp.sum(eq, axis=0)
    scores = score_ref[:, :, 0]
    focus_ref[:, :] = jnp.max(scores, axis=(0, 1), keepdims=True)


def kernel(att_ws):
    a = att_ws.reshape(LH, L, T)
    nb = PART // SCAN_BLOCK
    outs = pl.pallas_call(
        _scan_kernel,
        grid=(nb,),
        in_specs=[
            pl.BlockSpec((SCAN_BLOCK, L, T),
                         lambda i, _k=k, _nb=nb: (i + _k * _nb, 0, 0))
            for k in range(NSTREAM)
        ],
        out_specs=[
            pl.BlockSpec((SCAN_BLOCK, 1, 1), lambda i: (i, 0, 0))
            for _ in range(NSTREAM)
        ],
        out_shape=[
            jax.ShapeDtypeStruct((PART, 1, 1), jnp.float32)
            for _ in range(NSTREAM)
        ],
    )(*([a] * NSTREAM))
    scores = jnp.concatenate(outs, axis=0)
    widx = jnp.argmax(scores.reshape(LH)).astype(jnp.int32).reshape(1)
    durations, focus = pl.pallas_call(
        _finalize_kernel,
        grid_spec=pltpu.PrefetchScalarGridSpec(
            num_scalar_prefetch=1,
            grid=(1,),
            in_specs=[
                pl.BlockSpec((1, L, T), lambda i, w: (w[0], 0, 0)),
                pl.BlockSpec((LH, 1, 1), lambda i, w: (0, 0, 0)),
            ],
            out_specs=[
                pl.BlockSpec((1, T), lambda i, w: (0, 0)),
                pl.BlockSpec((1, 1), lambda i, w: (0, 0)),
            ],
        ),
        out_shape=[
            jax.ShapeDtypeStruct((1, T), jnp.int32),
            jax.ShapeDtypeStruct((1, 1), jnp.float32),
        ],
    )(widx, a, scores)
    return durations.reshape(T), focus.reshape(())


# speedup vs baseline: 1.1686x; 1.1686x over previous
import jax
import jax.numpy as jnp
from jax.experimental import pallas as pl
from jax.experimental.pallas import tpu as pltpu

LAYERS = 6
HEADS = 8
LH = LAYERS * HEADS
L = 2048
T = 512

SCAN_BLOCK = 4
NB = LH // SCAN_BLOCK


def _scan_kernel(x_ref, dur_ref, focus_ref, best_ref, hist_ref):
    i = pl.program_id(0)

    @pl.when(i == 0)
    def _init():
        best_ref[0] = -1.0

    x = x_ref[...]
    rmax = jnp.max(x, axis=-1, keepdims=True)
    scores = jnp.mean(rmax, axis=1, keepdims=True)[:, :, 0]
    blk_max = jnp.max(scores)
    improved = blk_max > best_ref[0]

    @pl.when(improved)
    def _update():
        best_ref[0] = blk_max
        iota_b = jax.lax.broadcasted_iota(jnp.int32, (SCAN_BLOCK, 1), 0)
        h = jnp.min(jnp.where(scores == blk_max, iota_b, SCAN_BLOCK))
        xh = x_ref[h]
        rh = jnp.max(xh, axis=-1, keepdims=True)
        iota_t = jax.lax.broadcasted_iota(jnp.int32, (L, T), 1)
        ridx = jnp.min(jnp.where(xh == rh, iota_t, T), axis=-1, keepdims=True)
        eq = (ridx == iota_t).astype(jnp.int32)
        hist_ref[0, :] = jnp.sum(eq, axis=0)

    @pl.when(i == NB - 1)
    def _finish():
        dur_ref[0, :] = hist_ref[0, :]
        focus_ref[...] = jnp.full((1, 1), best_ref[0], jnp.float32)


def kernel(att_ws):
    a = att_ws.reshape(LH, L, T)
    durations, focus = pl.pallas_call(
        _scan_kernel,
        grid=(NB,),
        in_specs=[pl.BlockSpec((SCAN_BLOCK, L, T), lambda i: (i, 0, 0))],
        out_specs=[
            pl.BlockSpec((1, T), lambda i: (0, 0)),
            pl.BlockSpec((1, 1), lambda i: (0, 0)),
        ],
        out_shape=[
            jax.ShapeDtypeStruct((1, T), jnp.int32),
            jax.ShapeDtypeStruct((1, 1), jnp.float32),
        ],
        scratch_shapes=[
            pltpu.SMEM((1,), jnp.float32),
            pltpu.VMEM((1, T), jnp.int32),
        ],
    )(a)
    return durations.reshape(T), focus.reshape(())
